# Initial kernel scaffold; baseline (speedup 1.0000x reference)
import functools

import jax
import jax.numpy as jnp
from jax import lax
from jax.experimental import pallas as pl
from jax.experimental.pallas import tpu as pltpu
from jax.experimental.pallas import tpu_sc as plsc

N = 10000
NP = 10240
E = 320000
D = 64
NT = 16
NODES_PER_TILE = NP // NT
SB = 2
SB_NODES = NODES_PER_TILE // SB
EDGES_PER_TILE = E // NT
CHUNK = 125
NCHUNK = EDGES_PER_TILE // CHUNK
NITER = 10


def _mlp(h_pad, W0, b0, W1, b1, W2, b2):
    rows = 640
    grid = NP // rows

    def body(h_ref, w0, b0r, w1, b1r, w2, b2r, out):
        x = jnp.maximum(jnp.dot(h_ref[...], w0[...],
                                preferred_element_type=jnp.float32) + b0r[...], 0.0)
        x = jnp.maximum(jnp.dot(x, w1[...],
                                preferred_element_type=jnp.float32) + b1r[...], 0.0)
        out[...] = jnp.dot(x, w2[...],
                           preferred_element_type=jnp.float32) + b2r[...]

    return pl.pallas_call(
        body,
        grid=(grid,),
        in_specs=[
            pl.BlockSpec((rows, 128), lambda i: (i, 0)),
            pl.BlockSpec((128, 128), lambda i: (0, 0)),
            pl.BlockSpec((1, 128), lambda i: (0, 0)),
            pl.BlockSpec((128, 128), lambda i: (0, 0)),
            pl.BlockSpec((1, 128), lambda i: (0, 0)),
            pl.BlockSpec((128, D), lambda i: (0, 0)),
            pl.BlockSpec((1, D), lambda i: (0, 0)),
        ],
        out_specs=pl.BlockSpec((rows, D), lambda i: (i, 0)),
        out_shape=jax.ShapeDtypeStruct((NP, D), jnp.float32),
    )(h_pad, W0, b0.reshape(1, 128), W1, b1.reshape(1, 128),
      W2, b2.reshape(1, D))


def _vrsqrt(d):
    i = lax.bitcast_convert_type(d, jnp.int32)
    i = jnp.int32(0x5F3759DF) - lax.shift_right_logical(i, 1)
    y = lax.bitcast_convert_type(i, jnp.float32)
    for _ in range(3):
        y = y * (1.5 - 0.5 * d * y * y)
    return y


def _sc_propagate(lp, src3, dst3, wh, bhp):
    mesh = plsc.VectorSubcoreMesh(core_axis_name="c", subcore_axis_name="s")

    @functools.partial(
        pl.kernel,
        out_type=[
            jax.ShapeDtypeStruct((NP, D), jnp.float32),
            jax.ShapeDtypeStruct((NP,), jnp.float32),
            jax.ShapeDtypeStruct((NP,), jnp.float32),
        ],
        mesh=mesh,
        scratch_types=[
            pltpu.VMEM_SHARED((NP, D), jnp.float32),
            pltpu.VMEM_SHARED((NP, D), jnp.float32),
            pltpu.VMEM_SHARED((NP,), jnp.float32),
            pltpu.VMEM((NCHUNK, CHUNK), jnp.int32),
            pltpu.VMEM((NCHUNK, CHUNK), jnp.int32),
            pltpu.VMEM((CHUNK, D), jnp.float32),
            pltpu.VMEM((SB_NODES, D), jnp.float32),
            pltpu.VMEM((SB_NODES, D), jnp.float32),
            pltpu.VMEM((SB_NODES, D), jnp.float32),
            pltpu.VMEM((128,), jnp.float32),
            pltpu.VMEM((NODES_PER_TILE,), jnp.float32),
            pltpu.VMEM((NODES_PER_TILE,), jnp.float32),
            pltpu.VMEM((NODES_PER_TILE,), jnp.float32),
            pltpu.VMEM((NODES_PER_TILE,), jnp.float32),
            pltpu.VMEM((NODES_PER_TILE,), jnp.float32),
            pltpu.VMEM((NODES_PER_TILE,), jnp.float32),
            pltpu.VMEM((NODES_PER_TILE,), jnp.float32),
            pltpu.VMEM((D,), jnp.float32),
            pltpu.VMEM((8,), jnp.float32),
            pltpu.SemaphoreType.DMA,
        ],
    )
    def sck(lp_hbm, src_hbm, dst_hbm, wh_hbm, bh_hbm,
            prop_hbm, steps_hbm, rem_hbm,
            y_sp, a_sp, deg_sp, src_v, dst_v, rows_v, A_v, y_v, zero_v,
            ones_v, dinv_v, rdinv_v, steps_v, sum_v, cont_v, z_v, p_v,
            wh_v, bh_v, sem):
        cid = lax.axis_index("c")
        tid = lax.axis_index("s")

        @pl.when(cid == 0)
        def _():
            base = tid * NODES_PER_TILE
            zero16 = jnp.zeros((16,), jnp.float32)
            one16 = jnp.ones((16,), jnp.float32)

            def fill_zero(i, _):
                for j in range(D // 16):
                    zero_v[i, pl.ds(16 * j, 16)] = zero16
                return 0
            lax.fori_loop(0, SB_NODES, fill_zero, 0)
            for k in range(8):
                ones_v[pl.ds(16 * k, 16)] = one16

            def fill_scal(k, _):
                idx = pl.ds(16 * k, 16)
                steps_v[idx] = one16
                sum_v[idx] = zero16
                cont_v[idx] = one16
                dinv_v[idx] = one16
                return 0
            lax.fori_loop(0, NODES_PER_TILE // 16, fill_scal, 0)

            pltpu.sync_copy(src_hbm.at[tid], src_v)
            pltpu.sync_copy(dst_hbm.at[tid], dst_v)
            pltpu.sync_copy(wh_hbm, wh_v)
            pltpu.sync_copy(bh_hbm, bh_v)

            pltpu.sync_copy(dinv_v, deg_sp.at[pl.ds(base, NODES_PER_TILE)])
            for sb in range(SB):
                pltpu.sync_copy(
                    zero_v, a_sp.at[pl.ds(base + sb * SB_NODES, SB_NODES)])
            plsc.subcore_barrier()

            def deg_body(j, _):
                pltpu.sync_copy(ones_v.at[pl.ds(0, CHUNK)],
                                deg_sp.at[dst_v.at[j]], add=True)
                return 0
            lax.fori_loop(0, NCHUNK, deg_body, 0)
            plsc.subcore_barrier()

            pltpu.sync_copy(deg_sp.at[pl.ds(base, NODES_PER_TILE)], rdinv_v)

            def dinv_body(k, _):
                idx = pl.ds(16 * k, 16)
                d = rdinv_v[idx]
                r = _vrsqrt(d)
                dinv_v[idx] = r
                rdinv_v[idx] = d * r
                return 0
            lax.fori_loop(0, NODES_PER_TILE // 16, dinv_body, 0)

            for sb in range(SB):
                sbase = base + sb * SB_NODES

                def y0_body(i, _):
                    dv = dinv_v[sb * SB_NODES + i]
                    for j in range(D // 16):
                        idx = pl.ds(16 * j, 16)
                        y_v[i, idx] = y_v[i, idx] * dv
                    return 0

                pltpu.sync_copy(lp_hbm.at[pl.ds(sbase, SB_NODES)], y_v)
                lax.fori_loop(0, SB_NODES, y0_body, 0)
                pltpu.sync_copy(y_v, y_sp.at[pl.ds(sbase, SB_NODES)])
            plsc.subcore_barrier()

            def iteration(it, _):
                def edge_body(j, _):
                    pltpu.make_async_copy(
                        y_sp.at[src_v.at[j]], rows_v, sem).start()
                    pltpu.make_async_copy(
                        y_sp.at[src_v.at[j]], rows_v, sem).wait()
                    pltpu.sync_copy(rows_v, a_sp.at[dst_v.at[j]], add=True)
                    return 0
                lax.fori_loop(0, NCHUNK, edge_body, 0)
                plsc.subcore_barrier()

                for sb in range(SB):
                    sbase = base + sb * SB_NODES
                    pltpu.sync_copy(a_sp.at[pl.ds(sbase, SB_NODES)], A_v)
                    pltpu.sync_copy(y_sp.at[pl.ds(sbase, SB_NODES)], y_v)
                    pltpu.sync_copy(zero_v, a_sp.at[pl.ds(sbase, SB_NODES)])

                    def pass1(i, _):
                        dv = dinv_v[sb * SB_NODES + i]
                        acc = zero16
                        for j in range(D // 16):
                            idx = pl.ds(16 * j, 16)
                            pr = (A_v[i, idx] + y_v[i, idx]) * dv
                            A_v[i, idx] = pr
                            acc = acc + pr * wh_v[idx]
                        z_v[sb * SB_NODES + i] = jnp.sum(acc)
                        return 0
                    lax.fori_loop(0, SB_NODES, pass1, 0)

                    def pass2(k, _):
                        idx = pl.ds(sb * SB_NODES + 16 * k, 16)
                        z = z_v[idx] + bh_v[0]
                        hh = 1.0 / (1.0 + jnp.exp(-z))
                        sh = sum_v[idx]
                        st = steps_v[idx]
                        co = cont_v[idx]
                        prob = jnp.where(((sh + hh) < 0.99) & (co > 0.0),
                                         1.0, 0.0)
                        st = st + prob
                        sh = sh + prob * hh
                        fin = jnp.where(st < float(NITER), 1.0, 0.0)
                        cond = prob * fin
                        p = jnp.where(cond > 0.0, sh, 1.0 - sh)
                        p_eff = jnp.where(co > 0.0, p, 1.0)
                        steps_v[idx] = st
                        sum_v[idx] = sh
                        cont_v[idx] = prob
                        p_v[idx] = p_eff
                        return 0
                    lax.fori_loop(0, SB_NODES // 16, pass2, 0)

                    def pass3(i, _):
                        pe = p_v[sb * SB_NODES + i]
                        rd = rdinv_v[sb * SB_NODES + i]
                        dv = dinv_v[sb * SB_NODES + i]
                        for j in range(D // 16):
                            idx = pl.ds(16 * j, 16)
                            pr = A_v[i, idx]
                            op = y_v[i, idx] * rd
                            prop = pe * pr + (1.0 - pe) * op
                            A_v[i, idx] = prop
                            y_v[i, idx] = prop * dv
                        return 0
                    lax.fori_loop(0, SB_NODES, pass3, 0)

                    pltpu.sync_copy(y_v, y_sp.at[pl.ds(sbase, SB_NODES)])

                    @pl.when(it == NITER - 1)
                    def _():
                        pltpu.sync_copy(
                            A_v, prop_hbm.at[pl.ds(sbase, SB_NODES)])

                plsc.subcore_barrier()
                return 0

            lax.fori_loop(0, NITER, iteration, 0)

            def rem_body(k, _):
                idx = pl.ds(16 * k, 16)
                p_v[idx] = 1.0 - sum_v[idx]
                return 0
            lax.fori_loop(0, NODES_PER_TILE // 16, rem_body, 0)
            pltpu.sync_copy(steps_v, steps_hbm.at[pl.ds(base, NODES_PER_TILE)])
            pltpu.sync_copy(p_v, rem_hbm.at[pl.ds(base, NODES_PER_TILE)])

    return sck(lp, src3, dst3, wh, bhp)


def kernel(g, h, e, snorm_n, snorm_e, W0, b0, W1, b1, W2, b2, Wh, bh):
    h_pad = jnp.pad(h, ((0, NP - N), (0, 0)))
    lp = _mlp(h_pad, W0, b0, W1, b1, W2, b2)
    src3 = g[0].reshape(NT, NCHUNK, CHUNK)
    dst3 = g[1].reshape(NT, NCHUNK, CHUNK)
    wh = Wh[:, 0]
    bhp = jnp.pad(bh, (0, 7))
    prop, steps, rem = _sc_propagate(lp, src3, dst3, wh, bhp)
    return prop[:N], steps[:N], rem[:N]


# speedup vs baseline: 9.0125x; 9.0125x over previous
import functools

import jax
import jax.numpy as jnp
from jax import lax
from jax.experimental import pallas as pl
from jax.experimental.pallas import tpu as pltpu
from jax.experimental.pallas import tpu_sc as plsc

N = 10000
NP = 10240
E = 320000
D = 64
NT = 16
NODES_PER_TILE = NP // NT
EDGES_PER_TILE = E // NT
CHUNK = 80
NCHUNK = EDGES_PER_TILE // CHUNK
NB = 25
NBLK = NCHUNK // NB
NITER = 10


def _mlp(h_pad, W0, b0, W1, b1, W2, b2):
    rows = 640
    grid = NP // rows

    def body(h_ref, w0, b0r, w1, b1r, w2, b2r, out):
        x = jnp.maximum(jnp.dot(h_ref[...], w0[...],
                                preferred_element_type=jnp.float32) + b0r[...], 0.0)
        x = jnp.maximum(jnp.dot(x, w1[...],
                                preferred_element_type=jnp.float32) + b1r[...], 0.0)
        out[...] = jnp.dot(x, w2[...],
                           preferred_element_type=jnp.float32) + b2r[...]

    return pl.pallas_call(
        body,
        grid=(grid,),
        in_specs=[
            pl.BlockSpec((rows, 128), lambda i: (i, 0)),
            pl.BlockSpec((128, 128), lambda i: (0, 0)),
            pl.BlockSpec((1, 128), lambda i: (0, 0)),
            pl.BlockSpec((128, 128), lambda i: (0, 0)),
            pl.BlockSpec((1, 128), lambda i: (0, 0)),
            pl.BlockSpec((128, D), lambda i: (0, 0)),
            pl.BlockSpec((1, D), lambda i: (0, 0)),
        ],
        out_specs=pl.BlockSpec((rows, D), lambda i: (i, 0)),
        out_shape=jax.ShapeDtypeStruct((NP, D), jnp.float32),
    )(h_pad, W0, b0.reshape(1, 128), W1, b1.reshape(1, 128),
      W2, b2.reshape(1, D))


def _sc_degree(dst3):
    mesh = plsc.VectorSubcoreMesh(core_axis_name="c", subcore_axis_name="s")

    @functools.partial(
        pl.kernel,
        out_type=jax.ShapeDtypeStruct((NP,), jnp.float32),
        mesh=mesh,
        scratch_types=[
            pltpu.VMEM_SHARED((NP,), jnp.float32),
            pltpu.VMEM((NB, CHUNK), jnp.int32),
            pltpu.VMEM((CHUNK,), jnp.float32),
            pltpu.VMEM((NODES_PER_TILE,), jnp.float32),
        ],
        compiler_params=pltpu.CompilerParams(use_tc_tiling_on_sc=False),
    )
    def degk(dst_hbm, deg_hbm, deg_sp, dblk_v, ones_v, init_v):
        cid = lax.axis_index("c")
        tid = lax.axis_index("s")

        @pl.when(cid == 0)
        def _():
            base = tid * NODES_PER_TILE
            one16 = jnp.ones((16,), jnp.float32)
            for k in range(CHUNK // 16):
                ones_v[pl.ds(16 * k, 16)] = one16

            def fill(k, _):
                init_v[pl.ds(16 * k, 16)] = one16
                return 0
            lax.fori_loop(0, NODES_PER_TILE // 16, fill, 0)
            pltpu.sync_copy(init_v, deg_sp.at[pl.ds(base, NODES_PER_TILE)])
            plsc.subcore_barrier()

            def blk(b, _):
                pltpu.sync_copy(dst_hbm.at[tid, pl.ds(b * NB, NB)], dblk_v)

                def body(j, _):
                    pltpu.sync_copy(ones_v, deg_sp.at[dblk_v.at[j]], add=True)
                    return 0
                lax.fori_loop(0, NB, body, 0)
                return 0
            lax.fori_loop(0, NBLK, blk, 0)
            plsc.subcore_barrier()
            pltpu.sync_copy(deg_sp.at[pl.ds(base, NODES_PER_TILE)],
                            deg_hbm.at[pl.ds(base, NODES_PER_TILE)])

    return degk(dst3)


def _sc_edge(y, src3, dst3):
    mesh = plsc.VectorSubcoreMesh(core_axis_name="c", subcore_axis_name="s")

    @functools.partial(
        pl.kernel,
        out_type=jax.ShapeDtypeStruct((NP, D), jnp.float32),
        mesh=mesh,
        scratch_types=[
            pltpu.VMEM_SHARED((NP, D), jnp.float32),
            pltpu.VMEM((NB, CHUNK), jnp.int32),
            pltpu.VMEM((NB, CHUNK), jnp.int32),
            pltpu.VMEM((CHUNK, D), jnp.float32),
            pltpu.VMEM((80, D), jnp.float32),
            pltpu.SemaphoreType.DMA,
        ],
        compiler_params=pltpu.CompilerParams(use_tc_tiling_on_sc=False),
    )
    def edgek(y_hbm, src_hbm, dst_hbm, a_hbm,
              a_sp, sblk_v, dblk_v, rows_v, zero_v, sem):
        cid = lax.axis_index("c")
        tid = lax.axis_index("s")

        @pl.when(cid == 0)
        def _():
            base = tid * NODES_PER_TILE
            zero16 = jnp.zeros((16,), jnp.float32)

            def fill_zero(i, _):
                for j in range(D // 16):
                    zero_v[i, pl.ds(16 * j, 16)] = zero16
                return 0
            lax.fori_loop(0, 80, fill_zero, 0)
            for hb in range(NODES_PER_TILE // 80):
                pltpu.sync_copy(zero_v, a_sp.at[pl.ds(base + hb * 80, 80)])
            plsc.subcore_barrier()

            def blk(b, _):
                pltpu.sync_copy(src_hbm.at[tid, pl.ds(b * NB, NB)], sblk_v)
                pltpu.sync_copy(dst_hbm.at[tid, pl.ds(b * NB, NB)], dblk_v)

                def body(j, _):
                    g = pltpu.make_async_copy(
                        y_hbm.at[sblk_v.at[j]], rows_v, sem)
                    g.start()
                    g.wait()
                    pltpu.sync_copy(rows_v, a_sp.at[dblk_v.at[j]], add=True)
                    return 0
                lax.fori_loop(0, NB, body, 0)
                return 0
            lax.fori_loop(0, NBLK, blk, 0)
            plsc.subcore_barrier()
            pltpu.sync_copy(a_sp.at[pl.ds(base, NODES_PER_TILE)],
                            a_hbm.at[pl.ds(base, NODES_PER_TILE)])

    return edgek(y, src3, dst3)


def kernel(g, h, e, snorm_n, snorm_e, W0, b0, W1, b1, W2, b2, Wh, bh):
    h_pad = jnp.pad(h, ((0, NP - N), (0, 0)))
    lp = _mlp(h_pad, W0, b0, W1, b1, W2, b2)
    src3 = g[0].reshape(NT, NCHUNK, CHUNK)
    dst3 = g[1].reshape(NT, NCHUNK, CHUNK)

    deg = _sc_degree(dst3)
    dinv = jnp.where(deg > 0, deg ** -0.5, 0.0)
    dcol = dinv[:, None]

    n = NP
    steps = jnp.ones(n, dtype=jnp.float32)
    sum_h = jnp.zeros(n, dtype=jnp.float32)
    continue_mask = jnp.ones(n, dtype=bool)
    prop = lp
    y = dcol * prop
    for _ in range(NITER):
        old_prop = prop
        A = _sc_edge(y, src3, dst3)
        praw = dcol * (A + y)
        hh = jax.nn.sigmoid(praw @ Wh + bh)[:, 0]
        prob_mask = ((sum_h + hh) < 0.99) & continue_mask
        prob_fmask = prob_mask.astype(jnp.float32)
        steps = steps + prob_fmask
        sum_h = sum_h + prob_fmask * hh
        final_iter = steps < NITER
        condition = prob_mask & final_iter
        p = jnp.where(condition, sum_h, 1.0 - sum_h)
        to_update = continue_mask.astype(jnp.float32)[:, None]
        prop = jnp.where(to_update == 0.0, praw,
                         p[:, None] * praw + (1.0 - p)[:, None] * old_prop)
        continue_mask = continue_mask & prob_mask
        y = dcol * prop
    reminders = 1.0 - sum_h
    return prop[:N], steps[:N], reminders[:N]
